# Initial kernel scaffold; baseline (speedup 1.0000x reference)
import jax
import jax.numpy as jnp
from jax.experimental import pallas as pl


def kernel(h, edge_index, params):
    raise NotImplementedError("write your pallas kernel here")


# speedup vs baseline: 4.1722x; 4.1722x over previous
import functools

import jax
import jax.numpy as jnp
from jax import lax
from jax.experimental import pallas as pl
from jax.experimental.pallas import tpu as pltpu
from jax.experimental.pallas import tpu_sc as plsc

N = 10000
E = 320000
D = 128
H = 128
O = 128

NC = 2
NS = 16
NW = NC * NS

CH = 128
NCHUNK = -(-E // CH)
CPW = -(-NCHUNK // NW)
E_PAD = NW * CPW * CH
N_PAD = 10240
RPT = N_PAD // NS
ROW_CHUNKS = RPT // CH


def _make_segsum():
    mesh = plsc.VectorSubcoreMesh(
        core_axis_name="c", subcore_axis_name="s", num_cores=NC, num_subcores=NS
    )

    @functools.partial(
        pl.kernel,
        out_type=jax.ShapeDtypeStruct((NC, N_PAD, D), jnp.float32),
        mesh=mesh,
        scratch_types=[
            pltpu.VMEM((CPW, CH), jnp.int32),
            pltpu.VMEM((CPW, CH), jnp.int32),
            pltpu.VMEM((CH, D), jnp.float32),
            pltpu.VMEM_SHARED((N_PAD, D), jnp.float32),
            pltpu.SemaphoreType.DMA,
        ],
    )
    def segsum(h_hbm, src_hbm, dst_hbm, out_hbm, src_v, dst_v, rows_v, acc_sh, sem):
        c = lax.axis_index("c")
        s = lax.axis_index("s")
        wid = c * NS + s

        pltpu.sync_copy(src_hbm.at[wid], src_v)
        pltpu.sync_copy(dst_hbm.at[wid], dst_v)

        zv = jnp.zeros((16,), jnp.float32)

        def zero_body(i, carry):
            for j in range(D // 16):
                rows_v[i, pl.ds(j * 16, 16)] = zv
            return carry

        lax.fori_loop(0, CH, zero_body, 0)
        for k in range(ROW_CHUNKS):
            pltpu.sync_copy(rows_v, acc_sh.at[pl.ds(s * RPT + k * CH, CH)])
        plsc.subcore_barrier()

        def body(j, carry):
            pltpu.async_copy(h_hbm.at[src_v.at[j]], rows_v, sem).wait()
            pltpu.sync_copy(rows_v, acc_sh.at[dst_v.at[j]], add=True)
            return carry

        lax.fori_loop(0, CPW, body, 0)
        plsc.subcore_barrier()

        for k in range(ROW_CHUNKS):
            r0 = s * RPT + k * CH
            pltpu.sync_copy(acc_sh.at[pl.ds(r0, CH)], out_hbm.at[c, pl.ds(r0, CH)])

    return segsum


_SEGSUM = None


def _segsum_call(h, src3, dst3):
    global _SEGSUM
    if _SEGSUM is None:
        _SEGSUM = _make_segsum()
    return _SEGSUM(h, src3, dst3)


def _dense_mid_body(h_ref, parts_ref, w1_ref, b1_ref, nw1_ref, nb1_ref, ns1_ref,
                    w2_ref, b2_ref, nw2_ref, nb2_ref, ns2_ref, out_ref):
    x = h_ref[...]
    neigh = parts_ref[0, :N, :] + parts_ref[1, :N, :]
    rst = x + neigh
    ones = jnp.full((1, N), 1.0 / N, dtype=jnp.float32)
    m = jnp.dot(rst, w1_ref[...], preferred_element_type=jnp.float32) + b1_ref[...]
    mean = jnp.dot(ones, m, preferred_element_type=jnp.float32)
    sub = m - mean * ns1_ref[...]
    var = jnp.dot(ones, sub * sub, preferred_element_type=jnp.float32)
    rstd = lax.rsqrt(var + 1e-6)
    m = jnp.maximum(nw1_ref[...] * sub * rstd + nb1_ref[...], 0.0)
    m = jnp.dot(m, w2_ref[...], preferred_element_type=jnp.float32) + b2_ref[...]
    mean2 = jnp.dot(ones, m, preferred_element_type=jnp.float32)
    sub2 = m - mean2 * ns2_ref[...]
    var2 = jnp.dot(ones, sub2 * sub2, preferred_element_type=jnp.float32)
    rstd2 = lax.rsqrt(var2 + 1e-6)
    out_ref[...] = jnp.maximum(nw2_ref[...] * sub2 * rstd2 + nb2_ref[...], 0.0)


def _dense_last_body(h_ref, parts_ref, w1_ref, b1_ref, nw1_ref, nb1_ref, ns1_ref,
                     w2_ref, b2_ref, nw2_ref, nb2_ref, ns2_ref,
                     pw_ref, pb_ref, out_ref):
    x = h_ref[...]
    neigh = parts_ref[0, :N, :] + parts_ref[1, :N, :]
    rst = x + neigh
    ones = jnp.full((1, N), 1.0 / N, dtype=jnp.float32)
    m = jnp.dot(rst, w1_ref[...], preferred_element_type=jnp.float32) + b1_ref[...]
    mean = jnp.dot(ones, m, preferred_element_type=jnp.float32)
    sub = m - mean * ns1_ref[...]
    var = jnp.dot(ones, sub * sub, preferred_element_type=jnp.float32)
    rstd = lax.rsqrt(var + 1e-6)
    m = jnp.maximum(nw1_ref[...] * sub * rstd + nb1_ref[...], 0.0)
    m = jnp.dot(m, w2_ref[...], preferred_element_type=jnp.float32) + b2_ref[...]
    mean2 = jnp.dot(ones, m, preferred_element_type=jnp.float32)
    sub2 = m - mean2 * ns2_ref[...]
    var2 = jnp.dot(ones, sub2 * sub2, preferred_element_type=jnp.float32)
    rstd2 = lax.rsqrt(var2 + 1e-6)
    h2 = jnp.maximum(nw2_ref[...] * sub2 * rstd2 + nb2_ref[...], 0.0) + x
    onesN = jnp.full((1, N), 1.0, dtype=jnp.float32)
    pooled = jnp.dot(onesN, h2, preferred_element_type=jnp.float32)
    out_ref[...] = (
        jnp.dot(pooled, pw_ref[...], preferred_element_type=jnp.float32) + pb_ref[...]
    )


def _dense_mid(h, parts, p):
    return pl.pallas_call(
        _dense_mid_body,
        out_shape=jax.ShapeDtypeStruct((N, H), jnp.float32),
    )(h, parts, p['W1'], p['b1'].reshape(1, H), p['mlp_nw'].reshape(1, H),
      p['mlp_nb'].reshape(1, H), p['mlp_ns'].reshape(1, H),
      p['W2'], p['b2'].reshape(1, H), p['app_nw'].reshape(1, H),
      p['app_nb'].reshape(1, H), p['app_ns'].reshape(1, H))


def _dense_last(h, parts, p, pw, pb):
    return pl.pallas_call(
        _dense_last_body,
        out_shape=jax.ShapeDtypeStruct((1, O), jnp.float32),
    )(h, parts, p['W1'], p['b1'].reshape(1, H), p['mlp_nw'].reshape(1, H),
      p['mlp_nb'].reshape(1, H), p['mlp_ns'].reshape(1, H),
      p['W2'], p['b2'].reshape(1, H), p['app_nw'].reshape(1, H),
      p['app_nb'].reshape(1, H), p['app_ns'].reshape(1, H),
      pw, pb.reshape(1, O))


def kernel(h, edge_index, params):
    src = edge_index[0]
    dst = edge_index[1]
    src3 = jnp.pad(src, (0, E_PAD - E)).reshape(NW, CPW, CH)
    dst3 = jnp.pad(dst, (0, E_PAD - E), constant_values=N).reshape(NW, CPW, CH)

    parts0 = _segsum_call(h, src3, dst3)
    h1 = _dense_mid(h, parts0, params['layers'][0])
    parts1 = _segsum_call(h1, src3, dst3)
    return _dense_last(h1, parts1, params['layers'][1],
                       params['pred_W'], params['pred_b'])
